# baseline (speedup 1.0000x reference)
import functools

import jax
import jax.numpy as jnp
from jax import lax
from jax.experimental import pallas as pl
from jax.experimental.pallas import tpu as pltpu
from jax.experimental.pallas import tpu_sc as plsc

_F32 = jnp.float32
_I32 = jnp.int32


def _dot(a, b):
    return jnp.dot(a, b, preferred_element_type=_F32)


def _lrelu(m):
    return jnp.where(m >= 0.0, m, 0.2 * m)


def _t(a):
    return jnp.swapaxes(a, 0, 1)


def _flatten_rows(a):
    H = a.shape[0]
    return jnp.concatenate([a[i:i + 1, :] for i in range(H)], axis=1)


def _onehot(iota, idx):
    return jnp.equal(iota, idx).astype(_F32)


def _proj_body(
    x1r, ei1, x2r, ei2, xlit, xreg, eill, eirl,
    Wl1, Wr1, att1, Wl2, Wr2, att2,
    SUM1, SUM2, R8, EXP1w, EXP2w, D1w, D2w, P16,
    Wp_l, bp_lv, Wp_r, bp_rv,
    as_llr, ad_llr, as_rlr, ad_rlr,
    xl1p, exb1, dnb1, exl1, s1p, d1p,
    xl2p, exb2, dnb2, exl2, s2p, d2p,
    h_l, h_r, exbll, dn16ll, sllp, dllp, exbrl, dn16rl, srlp, drlp,
):
    def pad_ei(ei_ref, s_ref, d_ref, E, Epad):
        z = jnp.zeros((Epad - E,), _I32)
        s_ref[0:E] = ei_ref[0, :]
        s_ref[E:Epad] = z
        d_ref[0:E] = ei_ref[1, :]
        d_ref[E:Epad] = z

    def store_rows(ref, rowsT, E, Epad, W):
        ref[0:E, :] = _t(rowsT)
        ref[E:Epad, :] = jnp.zeros((Epad - E, W), _F32)

    def gat_scores(xT, Wl, Wr, att, SUM, EXPw, Dw, N, E):
        xlT = _dot(_t(Wl), xT)
        xrT = _dot(_t(Wr), xT)
        attc = _t(_flatten_rows(att))
        tl = _lrelu(xlT + xrT) * attc
        ex_loopT = jnp.exp(_dot(SUM, tl))
        return xlT, xrT, ex_loopT

    x1T = _t(x1r[...])
    xlT1, xrT1, exl1T = gat_scores(x1T, Wl1[...], Wr1[...], att1[...],
                                   SUM1[...], EXP1w[...], D1w[...], 85, 680)
    exl1[...] = _t(exl1T)
    z4 = jnp.zeros((4, 85), _F32)
    xl1p[...] = _t(jnp.concatenate([xlT1[0:60], z4, xlT1[60:120], z4], axis=0))
    iota1 = lax.broadcasted_iota(jnp.int32, (85, 680), 0)
    Msrc1T = _onehot(iota1, ei1[0:1, :])
    Mdst1T = _onehot(iota1, ei1[1:2, :])
    GT1 = _dot(xlT1, Msrc1T)
    mT1 = GT1 + _dot(xrT1, Mdst1T)
    tT1 = _lrelu(mT1) * _t(_flatten_rows(att1[...]))
    exT1 = jnp.exp(_dot(SUM1[...], tT1))
    store_rows(exb1, _dot(EXP1w[...], exT1), 680, 1024, 128)
    store_rows(dnb1, _dot(D1w[...], exT1), 680, 1024, 128)
    pad_ei(ei1, s1p, d1p, 680, 1024)

    x2T = _t(x2r[...])
    xlT2, xrT2, exl2T = gat_scores(x2T, Wl2[...], Wr2[...], att2[...],
                                   SUM2[...], EXP2w[...], D2w[...], 438, 3504)
    exl2[...] = _t(exl2T)
    z6 = jnp.zeros((6, 438), _F32)
    z96 = jnp.zeros((96, 438), _F32)
    xl2p[...] = _t(jnp.concatenate(
        [xlT2[0:10], z6, xlT2[10:20], z6, z96], axis=0))
    iota2 = lax.broadcasted_iota(jnp.int32, (438, 3504), 0)
    Msrc2T = _onehot(iota2, ei2[0:1, :])
    Mdst2T = _onehot(iota2, ei2[1:2, :])
    GT2 = _dot(xlT2, Msrc2T)
    mT2 = GT2 + _dot(xrT2, Mdst2T)
    tT2 = _lrelu(mT2) * _t(_flatten_rows(att2[...]))
    exT2 = jnp.exp(_dot(SUM2[...], tT2))
    store_rows(exb2, _dot(EXP2w[...], exT2), 3504, 3584, 128)
    store_rows(dnb2, _dot(D2w[...], exT2), 3504, 3584, 128)
    pad_ei(ei2, s2p, d2p, 3504, 3584)

    bplc = _t(bp_lv[...].reshape(1, 128))
    bprc = _t(bp_rv[...].reshape(1, 128))
    h_lT = _dot(_t(Wp_l[...]), _t(xlit[...])) + bplc
    h_rT = _dot(_t(Wp_r[...]), _t(xreg[...])) + bprc
    h_l[...] = _t(h_lT)
    h_r[...] = _t(h_rT)
    a_sllT = _dot(R8[...], h_lT * _t(_flatten_rows(as_llr[...])))
    a_dllT = _dot(R8[...], h_lT * _t(_flatten_rows(ad_llr[...])))
    a_srlT = _dot(R8[...], h_rT * _t(_flatten_rows(as_rlr[...])))
    a_drlT = _dot(R8[...], h_lT * _t(_flatten_rows(ad_rlr[...])))

    def han_scores(a_sT, a_dT, ei, Ns, Nd, E):
        iota_s = lax.broadcasted_iota(jnp.int32, (Ns, E), 0)
        MsrcT = _onehot(iota_s, ei[0:1, :])
        iota_d = lax.broadcasted_iota(jnp.int32, (Nd, E), 0)
        MdstT = _onehot(iota_d, ei[1:2, :])
        eT = _lrelu(_dot(a_sT, MsrcT) + _dot(a_dT, MdstT))
        return jnp.exp(eT)

    exTll = han_scores(a_sllT, a_dllT, eill[...], 85, 85, 680)
    store_rows(exbll, _dot(jnp.swapaxes(R8[...], 0, 1), exTll), 680, 1024, 128)
    store_rows(dn16ll, _dot(P16[...], exTll), 680, 1024, 128)
    pad_ei(eill, sllp, dllp, 680, 1024)

    exTrl = han_scores(a_srlT, a_drlT, eirl[...], 60, 85, 1000)
    store_rows(exbrl, _dot(jnp.swapaxes(R8[...], 0, 1), exTrl), 1000, 1024, 128)
    store_rows(dn16rl, _dot(P16[...], exTrl), 1000, 1024, 128)
    pad_ei(eirl, srlp, drlp, 1000, 1024)


def _rows_fma(ra, rb, rc, CH, W16):
    def body(i, c):
        for ci in range(W16):
            o = ci * 16
            v = ra[i, pl.ds(o, 16)] * rb[i, pl.ds(o, 16)]
            if rc is not None:
                v = v + rc[i, pl.ds(o, 16)]
            ra[i, pl.ds(o, 16)] = v
        return c

    lax.fori_loop(0, CH, body, 0)


def _sc_body(
    xl1p, exb1, dnb1, s1p, d1p, xl2p, exb2, dnb2, s2p, d2p,
    h_l, h_r, exbll, dn16ll, sllp, dllp, exbrl, dn16rl, srlp, drlp,
    z88, z440,
    op1, op2, opll, dpll, oprl, dprl,
    ra64, rb64, rc64, dd64, ra224, rb224, rc224,
    srcv64, dstv64, srcv224, dstv224,
    sh_op1, sh_op2, sh_opll, sh_dpll, sh_oprl, sh_dprl,
):
    cid = lax.axis_index("c")
    sid = lax.axis_index("s")

    @pl.when(cid == 0)
    def _work():
        wid = sid

        @pl.when(sid == 0)
        def _z1():
            pltpu.sync_copy(z88, sh_op1)
        pltpu.sync_copy(s1p.at[pl.ds(wid * 64, 64)], srcv64)
        pltpu.sync_copy(d1p.at[pl.ds(wid * 64, 64)], dstv64)
        pltpu.sync_copy(xl1p.at[srcv64], ra64)
        pltpu.sync_copy(exb1.at[pl.ds(wid * 64, 64)], rb64)
        pltpu.sync_copy(dnb1.at[pl.ds(wid * 64, 64)], rc64)
        plsc.subcore_barrier()
        _rows_fma(ra64, rb64, rc64, 64, 8)
        pltpu.sync_copy(ra64, sh_op1.at[dstv64], add=True)
        plsc.subcore_barrier()

        @pl.when(sid == 0)
        def _o1():
            pltpu.sync_copy(sh_op1, op1)

        @pl.when(sid == 0)
        def _z2():
            pltpu.sync_copy(z440, sh_op2)
        pltpu.sync_copy(s2p.at[pl.ds(wid * 224, 224)], srcv224)
        pltpu.sync_copy(d2p.at[pl.ds(wid * 224, 224)], dstv224)
        pltpu.sync_copy(xl2p.at[srcv224], ra224)
        pltpu.sync_copy(exb2.at[pl.ds(wid * 224, 224)], rb224)
        pltpu.sync_copy(dnb2.at[pl.ds(wid * 224, 224)], rc224)
        plsc.subcore_barrier()
        _rows_fma(ra224, rb224, rc224, 224, 8)
        pltpu.sync_copy(ra224, sh_op2.at[dstv224], add=True)
        plsc.subcore_barrier()

        @pl.when(sid == 0)
        def _o2():
            pltpu.sync_copy(sh_op2, op2)

        @pl.when(sid == 0)
        def _z3():
            pltpu.sync_copy(z88, sh_opll)
            pltpu.sync_copy(z88, sh_dpll)
        pltpu.sync_copy(sllp.at[pl.ds(wid * 64, 64)], srcv64)
        pltpu.sync_copy(dllp.at[pl.ds(wid * 64, 64)], dstv64)
        pltpu.sync_copy(h_l.at[srcv64], ra64)
        pltpu.sync_copy(exbll.at[pl.ds(wid * 64, 64)], rb64)
        pltpu.sync_copy(dn16ll.at[pl.ds(wid * 64, 64)], dd64)
        plsc.subcore_barrier()
        _rows_fma(ra64, rb64, None, 64, 8)
        pltpu.sync_copy(ra64, sh_opll.at[dstv64], add=True)
        pltpu.sync_copy(dd64, sh_dpll.at[dstv64], add=True)
        plsc.subcore_barrier()

        @pl.when(sid == 0)
        def _o3():
            pltpu.sync_copy(sh_opll, opll)
            pltpu.sync_copy(sh_dpll, dpll)

        @pl.when(sid == 0)
        def _z4():
            pltpu.sync_copy(z88, sh_oprl)
            pltpu.sync_copy(z88, sh_dprl)
        pltpu.sync_copy(srlp.at[pl.ds(wid * 64, 64)], srcv64)
        pltpu.sync_copy(drlp.at[pl.ds(wid * 64, 64)], dstv64)
        pltpu.sync_copy(h_r.at[srcv64], ra64)
        pltpu.sync_copy(exbrl.at[pl.ds(wid * 64, 64)], rb64)
        pltpu.sync_copy(dn16rl.at[pl.ds(wid * 64, 64)], dd64)
        plsc.subcore_barrier()
        _rows_fma(ra64, rb64, None, 64, 8)
        pltpu.sync_copy(ra64, sh_oprl.at[dstv64], add=True)
        pltpu.sync_copy(dd64, sh_dprl.at[dstv64], add=True)
        plsc.subcore_barrier()

        @pl.when(sid == 0)
        def _o4():
            pltpu.sync_copy(sh_oprl, oprl)
            pltpu.sync_copy(sh_dprl, dprl)


def _reduce_body(
    op1, op2, opll, dpll, oprl, dprl,
    xl1p, xl2p, exl1, exl2,
    b1v, b2v, Wk, bkv, qv, Wh, bhv,
    y1o, y2o, y3o,
):
    P1 = op1[...][0:85, :]
    el1 = exl1[...]
    xl1 = xl1p[...][0:85, :]
    num0 = P1[:, 0:60] + el1[:, 0:1] * xl1[:, 0:60]
    num1 = P1[:, 64:124] + el1[:, 1:2] * xl1[:, 64:124]
    den0 = P1[:, 60:61] + el1[:, 0:1] + 1e-16
    den1 = P1[:, 61:62] + el1[:, 1:2] + 1e-16
    o1 = jnp.concatenate([num0 / den0, num1 / den1], axis=1)
    y1o[...] = jnp.maximum(o1 + b1v[...], 0.0)

    P2 = op2[...][0:438, :]
    el2 = exl2[...]
    xl2 = xl2p[...][0:438, :]
    n0 = P2[:, 0:10] + el2[:, 0:1] * xl2[:, 0:10]
    n1 = P2[:, 16:26] + el2[:, 1:2] * xl2[:, 16:26]
    d0 = P2[:, 32:33] + el2[:, 0:1] + 1e-16
    d1 = P2[:, 33:34] + el2[:, 1:2] + 1e-16
    o2 = jnp.concatenate([n0 / d0, n1 / d1], axis=1)
    y2o[...] = jnp.maximum(o2 + b2v[...], 0.0)

    def han_out(opref, dpref):
        P = opref[...][0:85, :]
        dd = dpref[...][0:85, 0:8]
        cols = [P[:, h * 16:(h + 1) * 16] / (dd[:, h:h + 1] + 1e-16)
                for h in range(8)]
        return jnp.maximum(jnp.concatenate(cols, axis=1), 0.0)

    o_llT = _t(han_out(opll, dpll))
    o_rlT = _t(han_out(oprl, dprl))

    bkc = _t(bkv[...].reshape(1, 128))
    qc = _t(qv[...].reshape(1, 128))
    k0 = jnp.tanh(_dot(_t(Wk[...]), o_llT) + bkc)
    k1 = jnp.tanh(_dot(_t(Wk[...]), o_rlT) + bkc)
    mean0 = jnp.sum(k0, axis=1, keepdims=True) * (1.0 / 85.0)
    mean1 = jnp.sum(k1, axis=1, keepdims=True) * (1.0 / 85.0)
    sc0 = jnp.sum(qc * mean0, axis=0, keepdims=True)
    sc1 = jnp.sum(qc * mean1, axis=0, keepdims=True)
    mx = jnp.maximum(sc0, sc1)
    e0 = jnp.exp(sc0 - mx)
    e1 = jnp.exp(sc1 - mx)
    inv = 1.0 / (e0 + e1)
    hT = (e0 * inv) * o_llT + (e1 * inv) * o_rlT
    y3T = _dot(_t(Wh[...]), hT)
    y3o[...] = jnp.maximum(_t(y3T) + bhv[...], 0.0)


def _final_body(x, Wf, bfv, o):
    o[...] = _dot(x[...], Wf[...]) + bfv[...]


def kernel(x1, edge_index1, x2, edge_index2, x_litter, x_region, ei_ll, ei_rl,
           Wl1, Wr1, att1, b1, Wl2, Wr2, att2, b2,
           Wp_l, bp_l, Wp_r, bp_r, as_ll, ad_ll, as_rl, ad_rl,
           Wk, bk, q, Wh, bh, Wf, bf):
    f32 = _F32
    SUM1 = jnp.kron(jnp.eye(2, dtype=f32), jnp.ones((1, 60), dtype=f32))
    SUM2 = jnp.kron(jnp.eye(2, dtype=f32), jnp.ones((1, 10), dtype=f32))
    R8 = jnp.kron(jnp.eye(8, dtype=f32), jnp.ones((1, 16), dtype=f32))
    EXP1w = jnp.kron(jnp.eye(2, dtype=f32), jnp.ones((64, 1), dtype=f32))
    EXP2w = jnp.concatenate(
        [jnp.kron(jnp.eye(2, dtype=f32), jnp.ones((16, 1), dtype=f32)),
         jnp.zeros((96, 2), dtype=f32)], axis=0)
    D1w = jnp.zeros((128, 2), f32).at[60, 0].set(1.0).at[61, 1].set(1.0)
    D2w = jnp.zeros((128, 2), f32).at[32, 0].set(1.0).at[33, 1].set(1.0)
    P16 = jnp.concatenate(
        [jnp.eye(8, dtype=f32), jnp.zeros((120, 8), f32)], axis=0)

    sds = jax.ShapeDtypeStruct
    (xl1p, exb1, dnb1, exl1, s1p, d1p,
     xl2p, exb2, dnb2, exl2, s2p, d2p,
     h_l, h_r, exbll, dn16ll, sllp, dllp,
     exbrl, dn16rl, srlp, drlp) = pl.pallas_call(
        _proj_body,
        out_shape=(
            sds((85, 128), f32), sds((1024, 128), f32), sds((1024, 128), f32),
            sds((85, 2), f32), sds((1024,), _I32), sds((1024,), _I32),
            sds((438, 128), f32), sds((3584, 128), f32), sds((3584, 128), f32),
            sds((438, 2), f32), sds((3584,), _I32), sds((3584,), _I32),
            sds((85, 128), f32), sds((60, 128), f32),
            sds((1024, 128), f32), sds((1024, 128), f32),
            sds((1024,), _I32), sds((1024,), _I32),
            sds((1024, 128), f32), sds((1024, 128), f32),
            sds((1024,), _I32), sds((1024,), _I32),
        ),
    )(x1, edge_index1, x2, edge_index2, x_litter, x_region, ei_ll, ei_rl,
      Wl1, Wr1, att1, Wl2, Wr2, att2,
      SUM1, SUM2, R8, EXP1w, EXP2w, D1w, D2w, P16,
      Wp_l, bp_l, Wp_r, bp_r, as_ll, ad_ll, as_rl, ad_rl)

    z88 = jnp.zeros((88, 128), f32)
    z440 = jnp.zeros((440, 128), f32)

    mesh = plsc.VectorSubcoreMesh(core_axis_name="c", subcore_axis_name="s")
    sc = functools.partial(
        pl.kernel, mesh=mesh,
        out_type=(
            pltpu.HBM((88, 128), f32), pltpu.HBM((440, 128), f32),
            pltpu.HBM((88, 128), f32), pltpu.HBM((88, 128), f32),
            pltpu.HBM((88, 128), f32), pltpu.HBM((88, 128), f32),
        ),
        scratch_types=[
            pltpu.VMEM((64, 128), f32), pltpu.VMEM((64, 128), f32),
            pltpu.VMEM((64, 128), f32), pltpu.VMEM((64, 128), f32),
            pltpu.VMEM((224, 128), f32), pltpu.VMEM((224, 128), f32),
            pltpu.VMEM((224, 128), f32),
            pltpu.VMEM((64,), _I32), pltpu.VMEM((64,), _I32),
            pltpu.VMEM((224,), _I32), pltpu.VMEM((224,), _I32),
            pltpu.MemorySpace.VMEM_SHARED((88, 128), f32),
            pltpu.MemorySpace.VMEM_SHARED((440, 128), f32),
            pltpu.MemorySpace.VMEM_SHARED((88, 128), f32),
            pltpu.MemorySpace.VMEM_SHARED((88, 128), f32),
            pltpu.MemorySpace.VMEM_SHARED((88, 128), f32),
            pltpu.MemorySpace.VMEM_SHARED((88, 128), f32),
        ],
    )(_sc_body)
    (op1, op2, opll, dpll, oprl, dprl) = sc(
        xl1p, exb1, dnb1, s1p, d1p, xl2p, exb2, dnb2, s2p, d2p,
        h_l, h_r, exbll, dn16ll, sllp, dllp, exbrl, dn16rl, srlp, drlp,
        z88, z440)

    y1, y2, y3 = pl.pallas_call(
        _reduce_body,
        out_shape=(
            sds((85, 120), f32), sds((438, 20), f32), sds((85, 120), f32),
        ),
    )(op1, op2, opll, dpll, oprl, dprl,
      xl1p, xl2p, exl1, exl2, b1, b2, Wk, bk, q, Wh, bh)

    xcat = jnp.concatenate(
        [y1.reshape(120, 85), y2.reshape(120, 73), y3.reshape(120, 85)], axis=1)

    return pl.pallas_call(
        _final_body,
        out_shape=sds((120, 5), f32),
    )(xcat, Wf, bf)
